# baseline (speedup 1.0000x reference)
import jax
import jax.numpy as jnp
import numpy as np
from jax import lax
from jax.experimental import pallas as pl
from jax.experimental.pallas import tpu as pltpu
from jax.experimental.pallas import tpu_sc as plsc

_B = 16384
_EMB = 64
_BN_EPS = 1e-5
_GLOBAL_MEAN = 3.5

_NC = 2
_NS = 16
_NW = _NC * _NS
_HB = _B // 2
_BPW = _HB // _NW
_CH = 128

_BLK = 4096


def _sc_gather_body(uidx_hbm, iidx_hbm, emb2_hbm, ubias_hbm, ibias_hbm,
                    kvec_hbm,
                    u_out, i_out, bias_out,
                    uraw_v, iraw_v, ub_v, ib_v, kv_v, p0_v, p1_v,
                    sem0, sem1, semb):
    wid = lax.axis_index("s") * _NC + lax.axis_index("c")
    base = wid * _BPW
    pltpu.sync_copy(uidx_hbm.at[pl.ds(base, _BPW)], uraw_v)
    pltpu.sync_copy(iidx_hbm.at[pl.ds(base, _BPW)], iraw_v)
    pltpu.sync_copy(kvec_hbm, kv_v)
    cb_u = pltpu.async_copy(ubias_hbm.at[uraw_v], ub_v, semb)
    cb_i = pltpu.async_copy(ibias_hbm.at[iraw_v], ib_v, semb)

    bufs = (p0_v, p1_v)
    sems = (sem0, sem1)
    n_c = _BPW // _CH
    jobs = [(uraw_v, u_out, c) for c in range(n_c)] + \
           [(iraw_v, i_out, c) for c in range(n_c)]

    inflight = [None, None]
    for j, (qref, out_hbm, c) in enumerate(jobs):
        slot = j % 2
        if inflight[slot] is not None:
            inflight[slot].wait()
            prev_out, prev_c = jobs[j - 2][1], jobs[j - 2][2]
            pltpu.sync_copy(bufs[slot],
                            prev_out.at[pl.ds(base + prev_c * _CH, _CH), :])
        inflight[slot] = pltpu.async_copy(
            emb2_hbm.at[qref.at[pl.ds(c * _CH, _CH)]], bufs[slot], sems[slot])
    for j in (len(jobs) - 2, len(jobs) - 1):
        slot = j % 2
        inflight[slot].wait()
        out_hbm, c = jobs[j][1], jobs[j][2]
        pltpu.sync_copy(bufs[slot],
                        out_hbm.at[pl.ds(base + c * _CH, _CH), :])

    cb_u.wait()
    cb_i.wait()
    kv = kv_v[...]
    for j in range(_BPW // 16):
        sl = pl.ds(j * 16, 16)
        ub_v[sl] = ub_v[sl] + ib_v[sl] + kv
    pltpu.sync_copy(ub_v, bias_out.at[pl.ds(base, _BPW)])


def _sc_gather(user_idx, item_idx, emb2, ubias1d, ibias1d, kvec):
    fn = pl.kernel(
        _sc_gather_body,
        out_type=[
            jax.ShapeDtypeStruct((_HB, 2 * _EMB), jnp.float32),
            jax.ShapeDtypeStruct((_HB, 2 * _EMB), jnp.float32),
            jax.ShapeDtypeStruct((_HB,), jnp.float32),
        ],
        mesh=plsc.VectorSubcoreMesh(core_axis_name="c", subcore_axis_name="s"),
        scratch_types=[
            pltpu.VMEM((_BPW,), jnp.int32),
            pltpu.VMEM((_BPW,), jnp.int32),
            pltpu.VMEM((_BPW,), jnp.float32),
            pltpu.VMEM((_BPW,), jnp.float32),
            pltpu.VMEM((16,), jnp.float32),
            pltpu.VMEM((_CH, 2 * _EMB), jnp.float32),
            pltpu.VMEM((_CH, 2 * _EMB), jnp.float32),
            pltpu.SemaphoreType.DMA,
            pltpu.SemaphoreType.DMA,
            pltpu.SemaphoreType.DMA,
        ],
    )
    return fn(user_idx, item_idx, emb2, ubias1d, ibias1d, kvec)


def _tc_body(u_ref, i_ref, f_ref, wide_ref, bias_ref,
             w1u_ref, w1i_ref, w1ftc_ref, c1_ref,
             w2_ref, c2_ref, wout_ref, wrow_ref, out_ref):
    f32 = jnp.float32
    tdot = lambda a, b: lax.dot_general(
        a, b, (((0,), (0,)), ((), ())), preferred_element_type=f32)
    h1 = jnp.dot(u_ref[...], w1u_ref[...], preferred_element_type=f32)
    h1 = h1 + jnp.dot(i_ref[...], w1i_ref[...], preferred_element_type=f32)
    h1 = h1 + tdot(f_ref[...], w1ftc_ref[...])
    h1 = jnp.maximum(h1 + c1_ref[...], 0.0)
    h2 = jnp.dot(h1, w2_ref[...], preferred_element_type=f32)
    h2 = jnp.maximum(h2 + c2_ref[...], 0.0)
    deep = jnp.dot(h2, wout_ref[...], preferred_element_type=f32)
    wide = tdot(wide_ref[...], wrow_ref[...])
    out_ref[...] = bias_ref[...] + deep[:, 0] + wide[:, 0]


def _tc_dense(u, i, feats, wide_features, bias,
              w1u, w1i, w1ftc, c1, w2, c2, woutT, wrowT):
    grid = (_HB // _BLK,)
    return pl.pallas_call(
        _tc_body,
        grid=grid,
        in_specs=[
            pl.BlockSpec((_BLK, 2 * _EMB), lambda i: (i, 0)),
            pl.BlockSpec((_BLK, 2 * _EMB), lambda i: (i, 0)),
            pl.BlockSpec((125, _BLK), lambda i: (0, i)),
            pl.BlockSpec((36, _BLK), lambda i: (0, i)),
            pl.BlockSpec((_BLK,), lambda i: (i,)),
            pl.BlockSpec((2 * _EMB, 256), lambda i: (0, 0)),
            pl.BlockSpec((2 * _EMB, 256), lambda i: (0, 0)),
            pl.BlockSpec((125, 256), lambda i: (0, 0)),
            pl.BlockSpec((1, 256), lambda i: (0, 0)),
            pl.BlockSpec((256, 128), lambda i: (0, 0)),
            pl.BlockSpec((1, 128), lambda i: (0, 0)),
            pl.BlockSpec((128, 1), lambda i: (0, 0)),
            pl.BlockSpec((36, 1), lambda i: (0, 0)),
        ],
        out_specs=pl.BlockSpec((_BLK,), lambda i: (i,)),
        out_shape=jax.ShapeDtypeStruct((_HB,), jnp.float32),
        compiler_params=pltpu.CompilerParams(
            dimension_semantics=("arbitrary",),
        ),
    )(u, i, feats, wide_features, bias,
      w1u, w1i, w1ftc, c1, w2, c2, woutT, wrowT)


def kernel(user_idx, item_idx, genre, tag, wide_features, deep_continuous,
           user_bias, item_bias, user_emb, item_emb, wide_W, wide_b,
           W1, b1, g1, be1, W2, b2, g2, be2, Wout, bout):
    inv = np.float32(1.0 / np.sqrt(1.0 + _BN_EPS))
    s1 = g1 * inv
    s2 = g2 * inv
    w1f = (W1 * s1[:, None]).T
    c1 = (b1 * s1 + be1)[None, :]
    w2f = (W2 * s2[:, None]).T
    c2 = (b2 * s2 + be2)[None, :]
    zpad = jnp.zeros((_EMB, 256), jnp.float32)
    w1u = jnp.concatenate([w1f[:_EMB], zpad], axis=0)
    w1i = jnp.concatenate([zpad, w1f[_EMB:2 * _EMB]], axis=0)
    w1ftc = w1f[128:253]
    kvec = jnp.broadcast_to(
        wide_b[0] + bout[0] + jnp.float32(_GLOBAL_MEAN), (16,))
    emb2 = jnp.concatenate([user_emb, item_emb], axis=1)
    feats = jnp.concatenate([genre, tag, deep_continuous], axis=1).T
    wideT = wide_features.T
    ub1 = user_bias[:, 0]
    ib1 = item_bias[:, 0]
    woutT = Wout.T
    wrowT = wide_W.T
    halves = []
    for h in range(2):
        sl = slice(h * _HB, (h + 1) * _HB)
        u, i, bias = _sc_gather(user_idx[sl], item_idx[sl], emb2,
                                ub1, ib1, kvec)
        halves.append(_tc_dense(u, i, feats[:, sl], wideT[:, sl], bias,
                                w1u, w1i, w1ftc, c1, w2f, c2,
                                woutT, wrowT))
    return jnp.concatenate(halves)
